# Initial kernel scaffold; baseline (speedup 1.0000x reference)
import jax
import jax.numpy as jnp
from jax.experimental import pallas as pl


def kernel(x, w1, b1, gamma, beta, w2, b2):
    raise NotImplementedError("write your pallas kernel here")


# speedup vs baseline: 2.1179x; 2.1179x over previous
import functools

import jax
import jax.numpy as jnp
from jax.experimental import pallas as pl
from jax.experimental.pallas import tpu as pltpu

_DN = (((1,), (1,)), ((), ()))


def _stats_kernel(x_ref, w1_ref, b1_ref, stats_ref):
    h = jax.lax.dot_general(x_ref[...], w1_ref[...], _DN,
                            preferred_element_type=jnp.float32)
    h = jnp.maximum(h + b1_ref[...], 0.0)
    s = jnp.sum(h, axis=0, keepdims=True)
    q = jnp.sum(h * h, axis=0, keepdims=True)
    stats_ref[0] = jnp.concatenate([s, q], axis=0)


def _out_kernel(x_ref, w1_ref, b1_ref, stats_ref, gamma_ref, beta_ref,
                w2_ref, b2_ref, o_ref, *, batch, eps):
    st = jnp.sum(stats_ref[...], axis=0)
    mean = st[0:1, :] * (1.0 / batch)
    var = jnp.maximum(st[1:2, :] * (1.0 / batch) - mean * mean, 0.0)
    scale = gamma_ref[...] * jax.lax.rsqrt(var + eps)
    shift = beta_ref[...] - mean * scale
    w2e = w2_ref[...] * scale
    b2e = jnp.sum(w2_ref[...] * shift) + b2_ref[0, 0]

    h = jax.lax.dot_general(x_ref[...], w1_ref[...], _DN,
                            preferred_element_type=jnp.float32)
    h = jnp.maximum(h + b1_ref[...], 0.0)
    y = jnp.sum(h * w2e, axis=1, keepdims=True) + b2e
    e = jnp.exp(-jnp.abs(y))
    o_ref[...] = jnp.where(y >= 0.0, 1.0, e) / (1.0 + e)


def _pick_tile(b, cap):
    tb = cap
    while tb > 1 and b % tb:
        tb //= 2
    return tb


def kernel(x, w1, b1, gamma, beta, w2, b2):
    B = x.shape[0]
    x = x.astype(jnp.float32)
    w1 = w1.astype(jnp.float32)
    b1r = b1.astype(jnp.float32).reshape(1, 6)
    gr = gamma.astype(jnp.float32).reshape(1, 6)
    br = beta.astype(jnp.float32).reshape(1, 6)
    w2r = w2.astype(jnp.float32).reshape(1, 6)
    b2r = b2.astype(jnp.float32).reshape(1, 1)

    tb1 = _pick_tile(B, 32768)
    nb1 = B // tb1
    tb2 = _pick_tile(B, 16384)
    nb2 = B // tb2

    parallel = pltpu.CompilerParams(dimension_semantics=("parallel",))

    stats = pl.pallas_call(
        _stats_kernel,
        out_shape=jax.ShapeDtypeStruct((nb1, 2, 6), jnp.float32),
        grid=(nb1,),
        in_specs=[pl.BlockSpec((tb1, 10), lambda i: (i, 0)),
                  pl.BlockSpec((6, 10), lambda i: (0, 0)),
                  pl.BlockSpec((1, 6), lambda i: (0, 0))],
        out_specs=pl.BlockSpec((1, 2, 6), lambda i: (i, 0, 0)),
        compiler_params=parallel,
    )(x, w1, b1r)

    out = pl.pallas_call(
        functools.partial(_out_kernel, batch=float(B), eps=1e-5),
        out_shape=jax.ShapeDtypeStruct((B, 1), jnp.float32),
        grid=(nb2,),
        in_specs=[pl.BlockSpec((tb2, 10), lambda i: (i, 0)),
                  pl.BlockSpec((6, 10), lambda i: (0, 0)),
                  pl.BlockSpec((1, 6), lambda i: (0, 0)),
                  pl.BlockSpec((nb1, 2, 6), lambda i: (0, 0, 0)),
                  pl.BlockSpec((1, 6), lambda i: (0, 0)),
                  pl.BlockSpec((1, 6), lambda i: (0, 0)),
                  pl.BlockSpec((1, 6), lambda i: (0, 0)),
                  pl.BlockSpec((1, 1), lambda i: (0, 0))],
        out_specs=pl.BlockSpec((tb2, 1), lambda i: (i, 0)),
        compiler_params=parallel,
    )(x, w1, b1r, stats, gr, br, w2r, b2r)

    return out


# speedup vs baseline: 4.0187x; 1.8974x over previous
import functools

import jax
import jax.numpy as jnp
from jax.experimental import pallas as pl
from jax.experimental.pallas import tpu as pltpu

F32 = jnp.float32


def _stats_kernel(x_ref, w1e_ref, b1t_ref, stats_ref):
    h = jnp.dot(x_ref[...], w1e_ref[...], preferred_element_type=F32)
    h = jnp.maximum(h + b1t_ref[...], 0.0)
    s = jnp.sum(h, axis=0, keepdims=True)
    q = jnp.sum(h * h, axis=0, keepdims=True)
    stats_ref[0] = jnp.concatenate([s, q], axis=0)


def _out_kernel(x_ref, w1e_ref, b1t_ref, w2t_ref, b2e_ref, o_ref, *, group):
    h = jnp.dot(x_ref[...], w1e_ref[...], preferred_element_type=F32)
    h = jnp.maximum(h + b1t_ref[...], 0.0)
    t = h * w2t_ref[...]
    k = jax.lax.broadcasted_iota(jnp.int32, (6 * group, group), 0)
    j = jax.lax.broadcasted_iota(jnp.int32, (6 * group, group), 1)
    g = jnp.where((k >= 6 * j) & (k < 6 * j + 6), 1.0, 0.0).astype(F32)
    y = jnp.dot(t, g, preferred_element_type=F32) + b2e_ref[...]
    e = jnp.exp(-jnp.abs(y))
    o_ref[...] = jnp.where(y >= 0.0, 1.0, e) / (1.0 + e)


def _pick_rows(rows, cap):
    r = cap
    while r > 1 and rows % r:
        r //= 2
    return r


def kernel(x, w1, b1, gamma, beta, w2, b2):
    B = x.shape[0]
    eps = 1e-5
    x = x.astype(F32)
    w1 = w1.astype(F32)
    G = 128 if B % 128 == 0 else (8 if B % 8 == 0 else 1)
    rows = B // G
    xr = x.reshape(rows, 10 * G)

    w1e = jnp.kron(jnp.eye(G, dtype=F32), w1.T)
    b1t = jnp.tile(b1.astype(F32).reshape(1, 6), (1, G))

    r1 = _pick_rows(rows, 512)
    nb1 = rows // r1
    parallel = pltpu.CompilerParams(dimension_semantics=("parallel",))

    stats = pl.pallas_call(
        _stats_kernel,
        out_shape=jax.ShapeDtypeStruct((nb1, 2, 6 * G), F32),
        grid=(nb1,),
        in_specs=[pl.BlockSpec((r1, 10 * G), lambda i: (i, 0)),
                  pl.BlockSpec((10 * G, 6 * G), lambda i: (0, 0)),
                  pl.BlockSpec((1, 6 * G), lambda i: (0, 0))],
        out_specs=pl.BlockSpec((1, 2, 6 * G), lambda i: (i, 0, 0)),
        compiler_params=parallel,
    )(xr, w1e, b1t)

    tot = jnp.sum(stats, axis=0).reshape(2, G, 6).sum(axis=1)
    mean = tot[0] / B
    var = jnp.maximum(tot[1] / B - mean * mean, 0.0)
    scale = gamma.astype(F32) / jnp.sqrt(var + eps)
    shift = beta.astype(F32) - mean * scale
    w2v = w2.astype(F32).reshape(6)
    w2t = jnp.tile((w2v * scale).reshape(1, 6), (1, G))
    b2e = (jnp.sum(w2v * shift) + b2.astype(F32).reshape(())).reshape(1, 1)

    r2 = _pick_rows(rows, 512)
    nb2 = rows // r2

    o8 = pl.pallas_call(
        functools.partial(_out_kernel, group=G),
        out_shape=jax.ShapeDtypeStruct((rows, G), F32),
        grid=(nb2,),
        in_specs=[pl.BlockSpec((r2, 10 * G), lambda i: (i, 0)),
                  pl.BlockSpec((10 * G, 6 * G), lambda i: (0, 0)),
                  pl.BlockSpec((1, 6 * G), lambda i: (0, 0)),
                  pl.BlockSpec((1, 6 * G), lambda i: (0, 0)),
                  pl.BlockSpec((1, 1), lambda i: (0, 0))],
        out_specs=pl.BlockSpec((r2, G), lambda i: (i, 0)),
        compiler_params=parallel,
    )(xr, w1e, b1t, w2t, b2e)

    return o8.reshape(B, 1)


# speedup vs baseline: 10.7434x; 2.6734x over previous
import jax
import jax.numpy as jnp
from jax import lax
from jax.experimental import pallas as pl
from jax.experimental.pallas import tpu as pltpu

F32 = jnp.float32


def _fc1_kernel(x_ref, w1t_ref, b1_ref, h_ref, stats_ref):
    xa = x_ref[0:8, :]
    xb = x_ref[8:10, :]
    hs = []
    for f in range(6):
        wa = w1t_ref[0:8, f:f + 1]
        wb = w1t_ref[8:10, f:f + 1]
        hf = (jnp.sum(xa * wa, axis=0, keepdims=True) +
              jnp.sum(xb * wb, axis=0, keepdims=True))
        hs.append(hf)
    h = jnp.concatenate(hs, axis=0)
    h = jnp.maximum(h + b1_ref[...], 0.0)
    h_ref[...] = h
    s = jnp.sum(h, axis=1, keepdims=True)
    q = jnp.sum(h * h, axis=1, keepdims=True)
    lane = lax.broadcasted_iota(jnp.int32, (6, 128), 1)
    stats_ref[0] = jnp.where(lane == 0, s, 0.0) + jnp.where(lane == 1, q, 0.0)


def _fc2_kernel(h_ref, w2e_ref, b2e_ref, o_ref):
    y = jnp.sum(h_ref[...] * w2e_ref[...], axis=0, keepdims=True) + b2e_ref[...]
    e = jnp.exp(-jnp.abs(y))
    o_ref[...] = jnp.where(y >= 0.0, 1.0, e) / (1.0 + e)


def _pick_tile(b, cap):
    tb = cap
    while tb > 1 and b % tb:
        tb //= 2
    return tb


def kernel(x, w1, b1, gamma, beta, w2, b2):
    B = x.shape[0]
    eps = 1e-5
    xt = x.astype(F32).T
    w1t = w1.astype(F32).T
    b1c = b1.astype(F32).reshape(6, 1)

    tb = _pick_tile(B, 65536)
    nb = B // tb
    parallel = pltpu.CompilerParams(dimension_semantics=("parallel",))

    h, stats = pl.pallas_call(
        _fc1_kernel,
        out_shape=[jax.ShapeDtypeStruct((6, B), F32),
                   jax.ShapeDtypeStruct((nb, 6, 128), F32)],
        grid=(nb,),
        in_specs=[pl.BlockSpec((10, tb), lambda i: (0, i)),
                  pl.BlockSpec((10, 6), lambda i: (0, 0)),
                  pl.BlockSpec((6, 1), lambda i: (0, 0))],
        out_specs=[pl.BlockSpec((6, tb), lambda i: (0, i)),
                   pl.BlockSpec((1, 6, 128), lambda i: (i, 0, 0))],
        compiler_params=parallel,
    )(xt, w1t, b1c)

    sums = jnp.sum(stats[:, :, 0], axis=0)
    sqs = jnp.sum(stats[:, :, 1], axis=0)
    mean = sums / B
    var = jnp.maximum(sqs / B - mean * mean, 0.0)
    scale = gamma.astype(F32) / jnp.sqrt(var + eps)
    shift = beta.astype(F32) - mean * scale
    w2v = w2.astype(F32).reshape(6)
    w2e = (w2v * scale).reshape(6, 1)
    b2e = (jnp.sum(w2v * shift) + b2.astype(F32).reshape(())).reshape(1, 1)

    out = pl.pallas_call(
        _fc2_kernel,
        out_shape=jax.ShapeDtypeStruct((1, B), F32),
        grid=(nb,),
        in_specs=[pl.BlockSpec((6, tb), lambda i: (0, i)),
                  pl.BlockSpec((6, 1), lambda i: (0, 0)),
                  pl.BlockSpec((1, 1), lambda i: (0, 0))],
        out_specs=pl.BlockSpec((1, tb), lambda i: (0, i)),
        compiler_params=parallel,
    )(h, w2e, b2e)

    return out.T


# speedup vs baseline: 10.7939x; 1.0047x over previous
import jax
import jax.numpy as jnp
from jax import lax
from jax.experimental import pallas as pl
from jax.experimental.pallas import tpu as pltpu

F32 = jnp.float32


def _fc1_kernel(x_ref, w1t_ref, b1_ref, h_ref, stats_ref):
    xa = x_ref[0:8, :]
    xb = x_ref[8:10, :]
    hs = []
    for f in range(6):
        wa = w1t_ref[0:8, f:f + 1]
        wb = w1t_ref[8:10, f:f + 1]
        hf = (jnp.sum(xa * wa, axis=0, keepdims=True) +
              jnp.sum(xb * wb, axis=0, keepdims=True))
        hs.append(hf)
    h = jnp.concatenate(hs, axis=0)
    h = jnp.maximum(h + b1_ref[...], 0.0)
    h_ref[...] = h.astype(jnp.bfloat16)
    s = jnp.sum(h, axis=1, keepdims=True)
    q = jnp.sum(h * h, axis=1, keepdims=True)
    lane = lax.broadcasted_iota(jnp.int32, (6, 128), 1)
    stats_ref[0] = jnp.where(lane == 0, s, 0.0) + jnp.where(lane == 1, q, 0.0)


def _fc2_kernel(h_ref, w2e_ref, b2e_ref, o_ref):
    y = (jnp.sum(h_ref[...].astype(F32) * w2e_ref[...], axis=0, keepdims=True)
         + b2e_ref[...])
    e = jnp.exp(-jnp.abs(y))
    o_ref[...] = jnp.where(y >= 0.0, 1.0, e) / (1.0 + e)


def _pick_tile(b, cap):
    tb = cap
    while tb > 1 and b % tb:
        tb //= 2
    return tb


def kernel(x, w1, b1, gamma, beta, w2, b2):
    B = x.shape[0]
    eps = 1e-5
    xt = x.astype(F32).T
    w1t = w1.astype(F32).T
    b1c = b1.astype(F32).reshape(6, 1)

    tb = _pick_tile(B, 65536)
    nb = B // tb
    parallel = pltpu.CompilerParams(dimension_semantics=("parallel",))

    h, stats = pl.pallas_call(
        _fc1_kernel,
        out_shape=[jax.ShapeDtypeStruct((6, B), jnp.bfloat16),
                   jax.ShapeDtypeStruct((nb, 6, 128), F32)],
        grid=(nb,),
        in_specs=[pl.BlockSpec((10, tb), lambda i: (0, i)),
                  pl.BlockSpec((10, 6), lambda i: (0, 0)),
                  pl.BlockSpec((6, 1), lambda i: (0, 0))],
        out_specs=[pl.BlockSpec((6, tb), lambda i: (0, i)),
                   pl.BlockSpec((1, 6, 128), lambda i: (i, 0, 0))],
        compiler_params=parallel,
    )(xt, w1t, b1c)

    sums = jnp.sum(stats[:, :, 0], axis=0)
    sqs = jnp.sum(stats[:, :, 1], axis=0)
    mean = sums / B
    var = jnp.maximum(sqs / B - mean * mean, 0.0)
    scale = gamma.astype(F32) / jnp.sqrt(var + eps)
    shift = beta.astype(F32) - mean * scale
    w2v = w2.astype(F32).reshape(6)
    w2e = (w2v * scale).reshape(6, 1)
    b2e = (jnp.sum(w2v * shift) + b2.astype(F32).reshape(())).reshape(1, 1)

    out = pl.pallas_call(
        _fc2_kernel,
        out_shape=jax.ShapeDtypeStruct((1, B), F32),
        grid=(nb,),
        in_specs=[pl.BlockSpec((6, tb), lambda i: (0, i)),
                  pl.BlockSpec((6, 1), lambda i: (0, 0)),
                  pl.BlockSpec((1, 1), lambda i: (0, 0))],
        out_specs=pl.BlockSpec((1, tb), lambda i: (0, i)),
        compiler_params=parallel,
    )(h, w2e, b2e)

    return out.T


# speedup vs baseline: 11.0231x; 1.0212x over previous
import functools

import jax
import jax.numpy as jnp
from jax import lax
from jax.experimental import pallas as pl
from jax.experimental.pallas import tpu as pltpu

F32 = jnp.float32


def _fc1_kernel(x_ref, w1t_ref, b1_ref, h_ref, stats_ref):
    xa = x_ref[0:8, :]
    xb = x_ref[8:10, :]
    hs = []
    for f in range(6):
        wa = w1t_ref[0:8, f:f + 1]
        wb = w1t_ref[8:10, f:f + 1]
        hf = (jnp.sum(xa * wa, axis=0, keepdims=True) +
              jnp.sum(xb * wb, axis=0, keepdims=True))
        hs.append(hf)
    h = jnp.concatenate(hs, axis=0)
    h = jnp.maximum(h + b1_ref[...], 0.0)
    h_ref[...] = h.astype(jnp.bfloat16)
    s = jnp.sum(h, axis=1, keepdims=True)
    q = jnp.sum(h * h, axis=1, keepdims=True)
    lane = lax.broadcasted_iota(jnp.int32, (6, 128), 1)
    stats_ref[0] = jnp.where(lane == 0, s, 0.0) + jnp.where(lane == 1, q, 0.0)


def _fc2_kernel(h_ref, stats_ref, g_ref, bt_ref, w2_ref, b2_ref, o_ref, *,
                batch, eps):
    tot = jnp.sum(stats_ref[...], axis=0)
    s = tot[:, 0:1]
    q = tot[:, 1:2]
    mean = s * (1.0 / batch)
    var = jnp.maximum(q * (1.0 / batch) - mean * mean, 0.0)
    scale = g_ref[...] * jax.lax.rsqrt(var + eps)
    shift = bt_ref[...] - mean * scale
    w2e = w2_ref[...] * scale
    b2e = jnp.sum(w2_ref[...] * shift) + b2_ref[0, 0]
    y = (jnp.sum(h_ref[...].astype(F32) * w2e, axis=0, keepdims=True)
         + b2e)
    o_ref[...] = 0.5 * jnp.tanh(0.5 * y) + 0.5


def _pick_tile(b, cap):
    tb = cap
    while tb > 1 and b % tb:
        tb //= 2
    return tb


def kernel(x, w1, b1, gamma, beta, w2, b2):
    B = x.shape[0]
    eps = 1e-5
    xt = x.astype(F32).T
    w1t = w1.astype(F32).T
    b1c = b1.astype(F32).reshape(6, 1)

    tb = _pick_tile(B, 65536)
    nb = B // tb
    parallel = pltpu.CompilerParams(dimension_semantics=("parallel",))

    h, stats = pl.pallas_call(
        _fc1_kernel,
        out_shape=[jax.ShapeDtypeStruct((6, B), jnp.bfloat16),
                   jax.ShapeDtypeStruct((nb, 6, 128), F32)],
        grid=(nb,),
        in_specs=[pl.BlockSpec((10, tb), lambda i: (0, i)),
                  pl.BlockSpec((10, 6), lambda i: (0, 0)),
                  pl.BlockSpec((6, 1), lambda i: (0, 0))],
        out_specs=[pl.BlockSpec((6, tb), lambda i: (0, i)),
                   pl.BlockSpec((1, 6, 128), lambda i: (i, 0, 0))],
        compiler_params=parallel,
    )(xt, w1t, b1c)

    gc = gamma.astype(F32).reshape(6, 1)
    btc = beta.astype(F32).reshape(6, 1)
    w2c = w2.astype(F32).reshape(6, 1)
    b2c = b2.astype(F32).reshape(1, 1)

    out = pl.pallas_call(
        functools.partial(_fc2_kernel, batch=float(B), eps=eps),
        out_shape=jax.ShapeDtypeStruct((1, B), F32),
        grid=(nb,),
        in_specs=[pl.BlockSpec((6, tb), lambda i: (0, i)),
                  pl.BlockSpec((nb, 6, 128), lambda i: (0, 0, 0)),
                  pl.BlockSpec((6, 1), lambda i: (0, 0)),
                  pl.BlockSpec((6, 1), lambda i: (0, 0)),
                  pl.BlockSpec((6, 1), lambda i: (0, 0)),
                  pl.BlockSpec((1, 1), lambda i: (0, 0))],
        out_specs=pl.BlockSpec((1, tb), lambda i: (0, i)),
        compiler_params=parallel,
    )(h, stats, gc, btc, w2c, b2c)

    return out.T


# speedup vs baseline: 11.1085x; 1.0077x over previous
import functools

import jax
import jax.numpy as jnp
from jax import lax
from jax.experimental import pallas as pl
from jax.experimental.pallas import tpu as pltpu

F32 = jnp.float32


def _fc1_kernel(x_ref, w1t_ref, b1_ref, h_ref, stats_ref):
    xa = x_ref[0:8, :]
    xb = x_ref[8:10, :]
    hs = []
    for f in range(6):
        wa = w1t_ref[0:8, f:f + 1]
        wb = w1t_ref[8:10, f:f + 1]
        hf = (jnp.sum(xa * wa, axis=0, keepdims=True) +
              jnp.sum(xb * wb, axis=0, keepdims=True))
        hs.append(hf)
    h = jnp.concatenate(hs, axis=0)
    h = jnp.maximum(h + b1_ref[...], 0.0)
    h_ref[...] = h.astype(jnp.bfloat16)
    s = jnp.sum(h, axis=1, keepdims=True)
    q = jnp.sum(h * h, axis=1, keepdims=True)
    lane = lax.broadcasted_iota(jnp.int32, (6, 128), 1)
    stats_ref[0] = jnp.where(lane == 0, s, 0.0) + jnp.where(lane == 1, q, 0.0)


def _fc2_kernel(h_ref, stats_ref, g_ref, bt_ref, w2_ref, b2_ref, o_ref, *,
                batch, eps):
    tot = jnp.sum(stats_ref[...], axis=0)
    s = tot[:, 0:1]
    q = tot[:, 1:2]
    mean = s * (1.0 / batch)
    var = jnp.maximum(q * (1.0 / batch) - mean * mean, 0.0)
    scale = g_ref[...] * jax.lax.rsqrt(var + eps)
    shift = bt_ref[...] - mean * scale
    w2e = w2_ref[...] * scale
    b2e = jnp.sum(w2_ref[...] * shift) + b2_ref[0, 0]
    y = (jnp.sum(h_ref[...].astype(F32) * w2e, axis=0, keepdims=True)
         + b2e)
    o_ref[...] = 0.5 * jnp.tanh(0.5 * y) + 0.5


def _pick_tile(b, cap):
    tb = cap
    while tb > 1 and b % tb:
        tb //= 2
    return tb


def kernel(x, w1, b1, gamma, beta, w2, b2):
    B = x.shape[0]
    eps = 1e-5
    xt = x.astype(F32).T
    w1t = w1.astype(F32).T
    b1c = b1.astype(F32).reshape(6, 1)

    tb = _pick_tile(B, 65536)
    nb = B // tb
    parallel = pltpu.CompilerParams(dimension_semantics=("parallel",))

    h, stats = pl.pallas_call(
        _fc1_kernel,
        out_shape=[jax.ShapeDtypeStruct((6, B), jnp.bfloat16),
                   jax.ShapeDtypeStruct((nb, 6, 128), F32)],
        grid=(nb,),
        in_specs=[pl.BlockSpec((10, tb), lambda i: (0, i)),
                  pl.BlockSpec((10, 6), lambda i: (0, 0)),
                  pl.BlockSpec((6, 1), lambda i: (0, 0))],
        out_specs=[pl.BlockSpec((6, tb), lambda i: (0, i)),
                   pl.BlockSpec((1, 6, 128), lambda i: (i, 0, 0))],
        compiler_params=parallel,
    )(xt, w1t, b1c)

    tb2 = _pick_tile(B, 131072)
    nb2 = B // tb2
    gc = gamma.astype(F32).reshape(6, 1)
    btc = beta.astype(F32).reshape(6, 1)
    w2c = w2.astype(F32).reshape(6, 1)
    b2c = b2.astype(F32).reshape(1, 1)

    out = pl.pallas_call(
        functools.partial(_fc2_kernel, batch=float(B), eps=eps),
        out_shape=jax.ShapeDtypeStruct((1, B), F32),
        grid=(nb2,),
        in_specs=[pl.BlockSpec((6, tb2), lambda i: (0, i)),
                  pl.BlockSpec((nb, 6, 128), lambda i: (0, 0, 0)),
                  pl.BlockSpec((6, 1), lambda i: (0, 0)),
                  pl.BlockSpec((6, 1), lambda i: (0, 0)),
                  pl.BlockSpec((6, 1), lambda i: (0, 0)),
                  pl.BlockSpec((1, 1), lambda i: (0, 0))],
        out_specs=pl.BlockSpec((1, tb2), lambda i: (0, i)),
        compiler_params=parallel,
    )(h, stats, gc, btc, w2c, b2c)

    return out.T
